# baseline (speedup 1.0000x reference)
import functools

import jax
import jax.numpy as jnp
from jax import lax
from jax.experimental import pallas as pl
from jax.experimental.pallas import tpu as pltpu
from jax.experimental.pallas import tpu_sc as plsc

N = 5000
K = 16
NPAD = 5120
BI = 1024
GRID = NPAD // BI
GR = 256
IDX_MASK = 8191
NEG_HALF_INV_SIG4 = -0.5 / (1.5 * 1.5)

NC, NS = 2, 16
NW = NC * NS
BW = NPAD // NW
VECS = BW // 16


def _bitonic_cleanup(v):
    n = len(v)
    d = n // 2
    while d >= 1:
        for i in range(n):
            if (i & d) == 0 and (i | d) < n:
                lo = jnp.minimum(v[i], v[i + d])
                hi = jnp.maximum(v[i], v[i + d])
                v[i], v[i + d] = lo, hi
        d //= 2
    return v


def _merge_full(a, b):
    return _bitonic_cleanup(a + b[::-1])


def _merge_low(a, b):
    m = len(a)
    c = [jnp.minimum(a[i], b[m - 1 - i]) if m - 1 - i < len(b) else a[i]
         for i in range(m)]
    return _bitonic_cleanup(c)


def _topk_body(m1_ref, m2_ref, ch1_ref, ch2t_ref, idx_ref, ch2p_ref):
    x2 = ch2t_ref[0:1, :]
    y2 = ch2t_ref[1:2, :]
    xy2 = x2 * y2
    x2p = m1_ref[0, 0] + m1_ref[0, 1] * y2 + m1_ref[1, 0] * x2 + m1_ref[1, 1] * xy2
    y2p = m2_ref[0, 0] + m2_ref[0, 1] * y2 + m2_ref[1, 0] * x2 + m2_ref[1, 1] * xy2

    @pl.when(pl.program_id(0) == 0)
    def _():
        ch2p_ref[0:1, :] = x2p
        ch2p_ref[1:2, :] = y2p

    lane = lax.broadcasted_iota(jnp.int32, (1, 128), 1)

    def group_body(g, carry):
        r0 = g * GR
        x1 = ch1_ref[pl.ds(r0, GR), 0:1]
        y1 = ch1_ref[pl.ds(r0, GR), 1:2]
        lists = []
        for c in range(NPAD // 128):
            xc = x2p[0:1, c * 128:(c + 1) * 128]
            yc = y2p[0:1, c * 128:(c + 1) * 128]
            dx = x1 - xc
            dy = y1 - yc
            d2 = dx * dx + dy * dy
            ki = lax.bitcast_convert_type(d2, jnp.int32)
            ki = jnp.bitwise_or(jnp.bitwise_and(ki, jnp.int32(-8192)),
                                lane + (c * 128))
            key = lax.bitcast_convert_type(ki, jnp.float32)
            if (c + 1) * 128 > N:
                key = jnp.where(lane + (c * 128) >= N,
                                jnp.float32(jnp.inf), key)
            lists.append([key])
        while len(lists) > 1:
            nxt = []
            for i in range(0, len(lists) - 1, 2):
                a, b = lists[i], lists[i + 1]
                if len(a) + len(b) > K:
                    nxt.append(_merge_low(a, b))
                else:
                    nxt.append(_merge_full(a, b))
            if len(lists) % 2:
                nxt.append(lists[-1])
            lists = nxt
        s = lists[0]
        cols = []
        for r in range(K):
            mn = jnp.min(s[0], axis=1, keepdims=True)
            cols.append(
                jnp.bitwise_and(lax.bitcast_convert_type(mn, jnp.int32),
                                IDX_MASK)
            )
            if r < K - 1:
                won = s[0] == mn
                depth = K - r
                for k in range(depth - 1):
                    s[k] = jnp.where(won, s[k + 1], s[k])
        idx_ref[pl.ds(r0, GR), :] = jnp.concatenate(cols, axis=1)
        return carry

    lax.fori_loop(0, BI // GR, group_body, 0)


_topk_call = pl.pallas_call(
    _topk_body,
    grid=(GRID,),
    in_specs=[
        pl.BlockSpec(memory_space=pltpu.SMEM),
        pl.BlockSpec(memory_space=pltpu.SMEM),
        pl.BlockSpec((BI, 2), lambda i: (i, 0)),
        pl.BlockSpec((2, NPAD), lambda i: (0, 0)),
    ],
    out_specs=[
        pl.BlockSpec((BI, K), lambda i: (i, 0)),
        pl.BlockSpec((2, NPAD), lambda i: (0, 0)),
    ],
    out_shape=[
        jax.ShapeDtypeStruct((NPAD, K), jnp.int32),
        jax.ShapeDtypeStruct((2, NPAD), jnp.float32),
    ],
)


def _sc_body(ch2px_hbm, ch2py_hbm, ch1f_hbm, j_hbm, s_hbm,
             xs_v, ys_v, jv, c1_v, sv, sem):
    wid = lax.axis_index("s") * NC + lax.axis_index("c")
    base = wid * BW
    copies = [
        pltpu.async_copy(ch2px_hbm, xs_v, sem),
        pltpu.async_copy(ch2py_hbm, ys_v, sem),
        pltpu.async_copy(ch1f_hbm.at[pl.ds(2 * base, 2 * BW)], c1_v, sem),
    ]
    copies += [
        pltpu.async_copy(j_hbm.at[pl.ds(a * NPAD + base, BW)],
                         jv.at[pl.ds(a * BW, BW)], sem)
        for a in range(K)
    ]
    for c in copies:
        c.wait()

    lane16 = lax.broadcasted_iota(jnp.int32, (16,), 0)

    def vbody(v, carry):
        off = v * 16
        i2 = (lane16 + off) * 2
        bx = plsc.load_gather(c1_v, [i2])
        by = plsc.load_gather(c1_v, [i2 + 1])
        acc = jnp.zeros((16,), jnp.float32)
        for a in range(K):
            idx = jv[pl.ds(a * BW + off, 16)]
            gx = plsc.load_gather(xs_v, [idx])
            gy = plsc.load_gather(ys_v, [idx])
            dx = bx - gx
            dy = by - gy
            acc = acc + jnp.exp((dx * dx + dy * dy) * NEG_HALF_INV_SIG4)
        sv[pl.ds(off, 16)] = acc * jnp.float32(1.0 / N)
        return carry

    lax.fori_loop(0, VECS, vbody, 0)
    pltpu.sync_copy(sv, s_hbm.at[pl.ds(base, BW)])


@functools.cache
def _get_sc_call():
    return pl.kernel(
        _sc_body,
        out_type=jax.ShapeDtypeStruct((NPAD,), jnp.float32),
        mesh=plsc.VectorSubcoreMesh(
            core_axis_name="c", subcore_axis_name="s", num_cores=NC, num_subcores=NS
        ),
        compiler_params=pltpu.CompilerParams(needs_layout_passes=False),
        scratch_types=[
            pltpu.VMEM((NPAD,), jnp.float32),
            pltpu.VMEM((NPAD,), jnp.float32),
            pltpu.VMEM((K * BW,), jnp.int32),
            pltpu.VMEM((2 * BW,), jnp.float32),
            pltpu.VMEM((BW,), jnp.float32),
            pltpu.SemaphoreType.DMA,
        ],
    )


def _logsum_body(s_ref, o_ref):
    s = s_ref[...]
    col = lax.broadcasted_iota(jnp.int32, (1, NPAD), 1)
    valid = jnp.logical_and(col < N, s != 0.0)
    t = jnp.where(valid, jnp.log(jnp.where(valid, s, 1.0)), 0.0)
    o_ref[0, 0] = -jnp.sum(t)


_logsum_call = pl.pallas_call(
    _logsum_body,
    in_specs=[pl.BlockSpec((1, NPAD), lambda: (0, 0))],
    out_specs=pl.BlockSpec(memory_space=pltpu.SMEM),
    out_shape=jax.ShapeDtypeStruct((1, 1), jnp.float32),
)


def kernel(ch1, ch2, M1, M2):
    pad = ((0, NPAD - N), (0, 0))
    ch1p = jnp.pad(ch1, pad)
    ch2t = jnp.pad(ch2, pad).T
    idx_pad, ch2p_t = _topk_call(M1, M2, ch1p, ch2t)
    j = idx_pad[:N].reshape(K, N)
    jflat = jnp.pad(j, ((0, 0), (0, NPAD - N))).reshape(-1)
    s = _get_sc_call()(ch2p_t[0], ch2p_t[1], ch1p.reshape(-1), jflat)
    out = _logsum_call(s.reshape(1, NPAD))
    return out[0, 0]


# speedup vs baseline: 1.0537x; 1.0537x over previous
import functools

import jax
import jax.numpy as jnp
from jax import lax
from jax.experimental import pallas as pl
from jax.experimental.pallas import tpu as pltpu
from jax.experimental.pallas import tpu_sc as plsc

N = 5000
K = 16
NPAD = 5120
BI = 256
GRID = NPAD // BI
GR = 256
IDX_MASK = 8191
NEG_HALF_INV_SIG4 = -0.5 / (1.5 * 1.5)

NC, NS = 2, 16
NW = NC * NS
BW = NPAD // NW
VECS = BW // 16


def _bitonic_cleanup(v):
    n = len(v)
    d = n // 2
    while d >= 1:
        for i in range(n):
            if (i & d) == 0 and (i | d) < n:
                lo = jnp.minimum(v[i], v[i + d])
                hi = jnp.maximum(v[i], v[i + d])
                v[i], v[i + d] = lo, hi
        d //= 2
    return v


def _merge_full(a, b):
    if len(a) == 1:
        return [jnp.minimum(a[0], b[0]), jnp.maximum(a[0], b[0])]
    ev = _merge_full(a[0::2], b[0::2])
    od = _merge_full(a[1::2], b[1::2])
    res = [ev[0]]
    for i in range(len(ev) - 1):
        res.append(jnp.minimum(od[i], ev[i + 1]))
        res.append(jnp.maximum(od[i], ev[i + 1]))
    res.append(od[-1])
    return res


def _merge_low(a, b):
    m = len(a)
    c = [jnp.minimum(a[i], b[m - 1 - i]) if m - 1 - i < len(b) else a[i]
         for i in range(m)]
    return _bitonic_cleanup(c)


def _topk_body(m1_ref, m2_ref, ch1_ref, ch2t_ref, idx_ref, ch2p_ref):
    x2 = ch2t_ref[0:1, :]
    y2 = ch2t_ref[1:2, :]
    xy2 = x2 * y2
    x2p = m1_ref[0, 0] + m1_ref[0, 1] * y2 + m1_ref[1, 0] * x2 + m1_ref[1, 1] * xy2
    y2p = m2_ref[0, 0] + m2_ref[0, 1] * y2 + m2_ref[1, 0] * x2 + m2_ref[1, 1] * xy2

    @pl.when(pl.program_id(0) == 0)
    def _():
        ch2p_ref[0:1, :] = x2p
        ch2p_ref[1:2, :] = y2p

    lane = lax.broadcasted_iota(jnp.int32, (1, 128), 1)

    def group_body(g, carry):
        r0 = g * GR
        x1 = ch1_ref[pl.ds(r0, GR), 0:1]
        y1 = ch1_ref[pl.ds(r0, GR), 1:2]
        lists = []
        for c in range(NPAD // 128):
            xc = x2p[0:1, c * 128:(c + 1) * 128]
            yc = y2p[0:1, c * 128:(c + 1) * 128]
            dx = x1 - xc
            dy = y1 - yc
            d2 = dx * dx + dy * dy
            ki = lax.bitcast_convert_type(d2, jnp.int32)
            ki = jnp.bitwise_or(jnp.bitwise_and(ki, jnp.int32(-8192)),
                                lane + (c * 128))
            key = lax.bitcast_convert_type(ki, jnp.float32)
            if (c + 1) * 128 > N:
                key = jnp.where(lane + (c * 128) >= N,
                                jnp.float32(jnp.inf), key)
            lists.append([key])
        while len(lists) > 1:
            nxt = []
            for i in range(0, len(lists) - 1, 2):
                a, b = lists[i], lists[i + 1]
                if len(a) + len(b) > K:
                    nxt.append(_merge_low(a, b))
                else:
                    nxt.append(_merge_full(a, b))
            if len(lists) % 2:
                nxt.append(lists[-1])
            lists = nxt
        s = lists[0]
        cols = []
        for r in range(K):
            mn = jnp.min(s[0], axis=1, keepdims=True)
            cols.append(
                jnp.bitwise_and(lax.bitcast_convert_type(mn, jnp.int32),
                                IDX_MASK)
            )
            if r < K - 1:
                won = s[0] == mn
                depth = K - r
                for k in range(depth - 1):
                    s[k] = jnp.where(won, s[k + 1], s[k])
        idx_ref[pl.ds(r0, GR), :] = jnp.concatenate(cols, axis=1)
        return carry

    lax.fori_loop(0, BI // GR, group_body, 0)


_topk_call = pl.pallas_call(
    _topk_body,
    grid=(GRID,),
    in_specs=[
        pl.BlockSpec(memory_space=pltpu.SMEM),
        pl.BlockSpec(memory_space=pltpu.SMEM),
        pl.BlockSpec((BI, 2), lambda i: (i, 0)),
        pl.BlockSpec((2, NPAD), lambda i: (0, 0)),
    ],
    out_specs=[
        pl.BlockSpec((BI, K), lambda i: (i, 0)),
        pl.BlockSpec((2, NPAD), lambda i: (0, 0)),
    ],
    out_shape=[
        jax.ShapeDtypeStruct((NPAD, K), jnp.int32),
        jax.ShapeDtypeStruct((2, NPAD), jnp.float32),
    ],
)


def _sc_body(ch2px_hbm, ch2py_hbm, ch1f_hbm, j_hbm, s_hbm,
             xs_v, ys_v, jv, c1_v, sv, sem):
    wid = lax.axis_index("s") * NC + lax.axis_index("c")
    base = wid * BW
    copies = [
        pltpu.async_copy(ch2px_hbm, xs_v, sem),
        pltpu.async_copy(ch2py_hbm, ys_v, sem),
        pltpu.async_copy(ch1f_hbm.at[pl.ds(2 * base, 2 * BW)], c1_v, sem),
    ]
    copies += [
        pltpu.async_copy(j_hbm.at[pl.ds(a * NPAD + base, BW)],
                         jv.at[pl.ds(a * BW, BW)], sem)
        for a in range(K)
    ]
    for c in copies:
        c.wait()

    lane16 = lax.broadcasted_iota(jnp.int32, (16,), 0)

    def vbody(v, carry):
        off = v * 16
        i2 = (lane16 + off) * 2
        bx = plsc.load_gather(c1_v, [i2])
        by = plsc.load_gather(c1_v, [i2 + 1])
        acc = jnp.zeros((16,), jnp.float32)
        for a in range(K):
            idx = jv[pl.ds(a * BW + off, 16)]
            gx = plsc.load_gather(xs_v, [idx])
            gy = plsc.load_gather(ys_v, [idx])
            dx = bx - gx
            dy = by - gy
            acc = acc + jnp.exp((dx * dx + dy * dy) * NEG_HALF_INV_SIG4)
        sv[pl.ds(off, 16)] = acc * jnp.float32(1.0 / N)
        return carry

    lax.fori_loop(0, VECS, vbody, 0)
    pltpu.sync_copy(sv, s_hbm.at[pl.ds(base, BW)])


@functools.cache
def _get_sc_call():
    return pl.kernel(
        _sc_body,
        out_type=jax.ShapeDtypeStruct((NPAD,), jnp.float32),
        mesh=plsc.VectorSubcoreMesh(
            core_axis_name="c", subcore_axis_name="s", num_cores=NC, num_subcores=NS
        ),
        compiler_params=pltpu.CompilerParams(needs_layout_passes=False),
        scratch_types=[
            pltpu.VMEM((NPAD,), jnp.float32),
            pltpu.VMEM((NPAD,), jnp.float32),
            pltpu.VMEM((K * BW,), jnp.int32),
            pltpu.VMEM((2 * BW,), jnp.float32),
            pltpu.VMEM((BW,), jnp.float32),
            pltpu.SemaphoreType.DMA,
        ],
    )


def _logsum_body(s_ref, o_ref):
    s = s_ref[...]
    col = lax.broadcasted_iota(jnp.int32, (1, NPAD), 1)
    valid = jnp.logical_and(col < N, s != 0.0)
    t = jnp.where(valid, jnp.log(jnp.where(valid, s, 1.0)), 0.0)
    o_ref[0, 0] = -jnp.sum(t)


_logsum_call = pl.pallas_call(
    _logsum_body,
    in_specs=[pl.BlockSpec((1, NPAD), lambda: (0, 0))],
    out_specs=pl.BlockSpec(memory_space=pltpu.SMEM),
    out_shape=jax.ShapeDtypeStruct((1, 1), jnp.float32),
)


def kernel(ch1, ch2, M1, M2):
    pad = ((0, NPAD - N), (0, 0))
    ch1p = jnp.pad(ch1, pad)
    ch2t = jnp.pad(ch2, pad).T
    idx_pad, ch2p_t = _topk_call(M1, M2, ch1p, ch2t)
    j = idx_pad[:N].reshape(K, N)
    jflat = jnp.pad(j, ((0, 0), (0, NPAD - N))).reshape(-1)
    s = _get_sc_call()(ch2p_t[0], ch2p_t[1], ch1p.reshape(-1), jflat)
    out = _logsum_call(s.reshape(1, NPAD))
    return out[0, 0]
